# Initial kernel scaffold; baseline (speedup 1.0000x reference)
import functools

import jax
import jax.numpy as jnp
from jax import lax
from jax.experimental import pallas as pl
from jax.experimental.pallas import tpu as pltpu
from jax.experimental.pallas import tpu_sc as plsc

N = 100000
D = 128
S = 512
NC = 2
NS = 16
NW = NC * NS
C = N // NW
T = 125
NT = C // T
CPAD = 3200
L = 16
NV = D // L

_NEG_INF = jnp.float32(-jnp.inf)


def _sc_body(x_hbm, batch_hbm, interior_hbm, bids_hbm, bvals_hbm,
             bbuf, xbuf, stage, cbuf, ibuf):
    wid = lax.axis_index("s") * NC + lax.axis_index("c")
    base = wid * C

    pltpu.sync_copy(batch_hbm.at[wid], bbuf)
    neg = jnp.full((L,), _NEG_INF, jnp.float32)
    for d in range(NV):
        cbuf[pl.ds(d * L, L)] = neg

    first_id = bbuf[0]

    def flush(cur, k, acc):
        for d in range(NV):
            stage[pl.ds(d * L, L)] = acc[d]

        @pl.when(k == 0)
        def _():
            pltpu.sync_copy(stage, bvals_hbm.at[wid, 0])

        @pl.when(k != 0)
        def _():
            pltpu.sync_copy(stage, interior_hbm.at[cur])

    def gap_fill(cur, nxt):
        def body(s, carry):
            pltpu.sync_copy(cbuf, interior_hbm.at[s])
            return carry
        lax.fori_loop(cur + 1, nxt, body, 0)

    def row_step(r, carry):
        cur, k = carry[0], carry[1]
        acc = carry[2:]
        b = bbuf[r]
        i = lax.rem(r, T)
        row = tuple(xbuf[i, pl.ds(d * L, L)] for d in range(NV))
        changed = b != cur

        @pl.when(changed)
        def _():
            flush(cur, k, acc)
            gap_fill(cur, b)

        new_acc = tuple(
            jnp.where(changed, rd, jnp.maximum(ad, rd))
            for ad, rd in zip(acc, row)
        )
        return (b, k + changed.astype(jnp.int32)) + new_acc

    def tile_step(t, carry):
        pltpu.sync_copy(x_hbm.at[pl.ds(base + t * T, T), :], xbuf)
        return lax.fori_loop(t * T, (t + 1) * T, row_step, carry)

    acc0 = tuple(jnp.full((L,), _NEG_INF, jnp.float32) for _ in range(NV))
    carry = lax.fori_loop(0, NT, tile_step, (first_id, jnp.int32(0)) + acc0)
    cur, k = carry[0], carry[1]
    acc = carry[2:]

    for d in range(NV):
        stage[pl.ds(d * L, L)] = acc[d]
    pltpu.sync_copy(stage, bvals_hbm.at[wid, 1])

    @pl.when(k == 0)
    def _():
        pltpu.sync_copy(stage, bvals_hbm.at[wid, 0])

    ids = lax.iota(jnp.int32, L)
    idvec = jnp.where(ids == 0, first_id, jnp.where(ids == 1, cur, 0))
    ibuf[...] = idvec
    pltpu.sync_copy(ibuf, bids_hbm.at[wid])


_sc_pool = pl.kernel(
    _sc_body,
    out_type=(
        jax.ShapeDtypeStruct((S, D), jnp.float32),
        jax.ShapeDtypeStruct((NW, L), jnp.int32),
        jax.ShapeDtypeStruct((NW, 2, D), jnp.float32),
    ),
    mesh=plsc.VectorSubcoreMesh(core_axis_name="c", subcore_axis_name="s"),
    scratch_types=[
        pltpu.VMEM((CPAD,), jnp.int32),
        pltpu.VMEM((T, D), jnp.float32),
        pltpu.VMEM((D,), jnp.float32),
        pltpu.VMEM((D,), jnp.float32),
        pltpu.VMEM((L,), jnp.int32),
    ],
)


def _merge_body(bids_smem, interior, bids_v, bvals, out_ref):
    ids = bids_v[...]
    lo = ids[:, 0:1]
    hi = ids[:, 1:2]
    cols = lax.broadcasted_iota(jnp.int32, (NW, S), 1)
    covered = jnp.any(jnp.logical_and(cols > lo, cols < hi), axis=0)
    out_ref[...] = jnp.where(covered[:, None], interior[...], _NEG_INF)
    for w in range(NW):
        for j in range(2):
            q = bids_smem[w, j]
            row = bvals[w, j, :].reshape(1, D)
            out_ref[pl.ds(q, 1), :] = jnp.maximum(out_ref[pl.ds(q, 1), :], row)


_merge = pl.pallas_call(
    _merge_body,
    out_shape=jax.ShapeDtypeStruct((S, D), jnp.float32),
    in_specs=[
        pl.BlockSpec(memory_space=pltpu.SMEM),
        pl.BlockSpec(memory_space=pltpu.VMEM),
        pl.BlockSpec(memory_space=pltpu.VMEM),
        pl.BlockSpec(memory_space=pltpu.VMEM),
    ],
)


@jax.jit
def kernel(x, batch):
    batch_pad = jnp.pad(batch.reshape(NW, C), ((0, 0), (0, CPAD - C)))
    interior, bids, bvals = _sc_pool(x, batch_pad)
    return _merge(bids, interior, bids, bvals)


# speedup vs baseline: 1.9988x; 1.9988x over previous
import functools

import jax
import jax.numpy as jnp
from jax import lax
from jax.experimental import pallas as pl
from jax.experimental.pallas import tpu as pltpu
from jax.experimental.pallas import tpu_sc as plsc

N = 100000
D = 128
S = 512
NC = 2
NS = 16
NW = NC * NS
C = N // NW
T = 125
NT = C // T
CPAD = 3200
L = 16
NV = D // L

_NEG_INF = float("-inf")


def _sc_body(x_hbm, batch_hbm, interior_hbm, bids_hbm, bvals_hbm,
             bbuf, xbuf, stage, cbuf, ibuf):
    wid = lax.axis_index("s") * NC + lax.axis_index("c")
    base = wid * C

    pltpu.sync_copy(batch_hbm.at[wid], bbuf)
    neg = jnp.full((L,), _NEG_INF, jnp.float32)
    for d in range(NV):
        cbuf[pl.ds(d * L, L)] = neg

    first_id = bbuf[pl.ds(0, L)][0]

    def flush(cur, k, acc):
        for d in range(NV):
            stage[pl.ds(d * L, L)] = acc[d]

        @pl.when(k == 0)
        def _():
            pltpu.sync_copy(stage, bvals_hbm.at[wid, 0])

        @pl.when(k != 0)
        def _():
            pltpu.sync_copy(stage, interior_hbm.at[cur])

    def gap_fill(cur, nxt):
        def body(s, carry):
            pltpu.sync_copy(cbuf, interior_hbm.at[s])
            return carry
        lax.fori_loop(cur + 1, nxt, body, 0)

    def row_step(r, carry):
        cur, k = carry[0], carry[1]
        acc = carry[2:]
        b = bbuf[pl.ds(r, L)][0]
        i = lax.rem(r, T)
        row = tuple(xbuf[i, pl.ds(d * L, L)] for d in range(NV))
        changed = b != cur

        @pl.when(changed)
        def _():
            flush(cur, k, acc)
            gap_fill(cur, b)

        new_acc = tuple(
            jnp.where(changed, rd, jnp.maximum(ad, rd))
            for ad, rd in zip(acc, row)
        )
        return (b, k + changed.astype(jnp.int32)) + new_acc

    def tile_step(t, carry):
        pltpu.sync_copy(x_hbm.at[pl.ds(base + t * T, T), :], xbuf)
        return lax.fori_loop(t * T, (t + 1) * T, row_step, carry)

    acc0 = tuple(jnp.full((L,), _NEG_INF, jnp.float32) for _ in range(NV))
    carry = lax.fori_loop(0, NT, tile_step, (first_id, jnp.int32(0)) + acc0)
    cur, k = carry[0], carry[1]
    acc = carry[2:]

    for d in range(NV):
        stage[pl.ds(d * L, L)] = acc[d]
    pltpu.sync_copy(stage, bvals_hbm.at[wid, 1])

    @pl.when(k == 0)
    def _():
        pltpu.sync_copy(stage, bvals_hbm.at[wid, 0])

    ids = lax.iota(jnp.int32, L)
    idvec = jnp.where(ids == 0, first_id, jnp.where(ids == 1, cur, 0))
    ibuf[...] = idvec
    pltpu.sync_copy(ibuf, bids_hbm.at[wid])


_sc_pool = pl.kernel(
    _sc_body,
    out_type=(
        jax.ShapeDtypeStruct((S, D), jnp.float32),
        jax.ShapeDtypeStruct((NW, L), jnp.int32),
        jax.ShapeDtypeStruct((NW, 2, D), jnp.float32),
    ),
    mesh=plsc.VectorSubcoreMesh(core_axis_name="c", subcore_axis_name="s"),
    compiler_params=pltpu.CompilerParams(use_tc_tiling_on_sc=False),
    scratch_types=[
        pltpu.VMEM((CPAD,), jnp.int32),
        pltpu.VMEM((T, D), jnp.float32),
        pltpu.VMEM((D,), jnp.float32),
        pltpu.VMEM((D,), jnp.float32),
        pltpu.VMEM((L,), jnp.int32),
    ],
)


def _merge_body(bids_smem, interior, bids_v, bvals, out_ref):
    del bids_v
    rows = lax.broadcasted_iota(jnp.int32, (S, D), 0)
    cov = jnp.zeros((S, D), jnp.bool_)
    for w in range(NW):
        lo = bids_smem[w, 0]
        hi = bids_smem[w, 1]
        cov = jnp.logical_or(cov, jnp.logical_and(rows > lo, rows < hi))
    out_ref[...] = jnp.where(cov, interior[...], _NEG_INF)
    for w in range(NW):
        for j in range(2):
            q = bids_smem[w, j]
            row = bvals[w, j, :].reshape(1, D)
            out_ref[pl.ds(q, 1), :] = jnp.maximum(out_ref[pl.ds(q, 1), :], row)


_merge = pl.pallas_call(
    _merge_body,
    out_shape=jax.ShapeDtypeStruct((S, D), jnp.float32),
    in_specs=[
        pl.BlockSpec(memory_space=pltpu.SMEM),
        pl.BlockSpec(memory_space=pltpu.VMEM),
        pl.BlockSpec(memory_space=pltpu.VMEM),
        pl.BlockSpec(memory_space=pltpu.VMEM),
    ],
)


@jax.jit
def kernel(x, batch):
    batch_pad = jnp.pad(batch.reshape(NW, C), ((0, 0), (0, CPAD - C)))
    interior, bids, bvals = _sc_pool(x, batch_pad)
    return _merge(bids, interior, bids, bvals)


# speedup vs baseline: 4.0718x; 2.0371x over previous
import jax
import jax.numpy as jnp
from jax import lax
from jax.experimental import pallas as pl
from jax.experimental.pallas import tpu as pltpu
from jax.experimental.pallas import tpu_sc as plsc

N = 100000
D = 128
S = 512
NC = 2
NS = 16
NW = NC * NS
G = 16
CW = 3136
STEP = 3128
TR = 448
NT = CW // TR
GPT = TR // G
L = 16
NV = D // L

_NEG_INF = float("-inf")


def _sc_body(x_hbm, batch_hbm, interior_hbm, bids_hbm, bvals_hbm,
             bbuf, xb0, xb1, stage, cbuf, ibuf, acc_ref, sem0, sem1):
    wid = lax.axis_index("s") * NC + lax.axis_index("c")
    base = jnp.minimum(wid * STEP, N - CW)

    pltpu.sync_copy(batch_hbm.at[pl.ds(base, CW)], bbuf.at[pl.ds(0, CW)])
    neg = jnp.full((L,), _NEG_INF, jnp.float32)
    for d in range(NV):
        cbuf[pl.ds(d * L, L)] = neg

    first_id = bbuf[pl.ds(0, L)][0]

    def flush(cur, k, acc):
        for d in range(NV):
            stage[pl.ds(d * L, L)] = acc[d]

        @pl.when(k == 0)
        def _():
            pltpu.sync_copy(stage, bvals_hbm.at[wid, 0])

        @pl.when(k != 0)
        def _():
            pltpu.sync_copy(stage, interior_hbm.at[cur])

    def gap_fill(cur, nxt):
        def body(s, carry):
            pltpu.sync_copy(cbuf, interior_hbm.at[s])
            return carry
        lax.fori_loop(cur + 1, nxt, body, 0)

    def make_group_body(xref, toff):
        def group_body(g, carry):
            goff = toff + g * G
            lrow = g * G
            idvec = bbuf[pl.ds(goff, L)]
            cur0 = carry[0]

            def fast(carry):
                acc = [acc_ref[pl.ds(d * L, L)] for d in range(NV)]
                for r in range(G):
                    for d in range(NV):
                        acc[d] = jnp.maximum(
                            acc[d], xref[lrow + r, pl.ds(d * L, L)])
                for d in range(NV):
                    acc_ref[pl.ds(d * L, L)] = acc[d]
                return carry

            def slow(carry):
                def row_step(r, rcarry):
                    cur, k = rcarry[0], rcarry[1]
                    acc = rcarry[2:]
                    b = bbuf[pl.ds(goff + r, L)][0]
                    changed = b != cur

                    @pl.when(changed)
                    def _():
                        flush(cur, k, acc)
                        gap_fill(cur, b)

                    row = tuple(
                        xref[lrow + r, pl.ds(d * L, L)] for d in range(NV))
                    new_acc = tuple(
                        jnp.where(changed, rd, jnp.maximum(ad, rd))
                        for ad, rd in zip(acc, row)
                    )
                    return (b, k + changed.astype(jnp.int32)) + new_acc

                acc0 = tuple(acc_ref[pl.ds(d * L, L)] for d in range(NV))
                out = lax.fori_loop(0, G, row_step, carry + acc0)
                for d in range(NV):
                    acc_ref[pl.ds(d * L, L)] = out[2 + d]
                return (out[0], out[1])

            return lax.cond(idvec[L - 1] == cur0, fast, slow, carry)
        return group_body

    for d in range(NV):
        acc_ref[pl.ds(d * L, L)] = neg
    carry = (first_id, jnp.int32(0))

    xbufs = (xb0, xb1)
    sems = (sem0, sem1)
    copies = [None, None]
    copies[0] = pltpu.async_copy(
        x_hbm.at[pl.ds(base, TR), :], xb0, sem0)
    for t in range(NT):
        if t + 1 < NT:
            nb = (t + 1) % 2
            copies[nb] = pltpu.async_copy(
                x_hbm.at[pl.ds(base + (t + 1) * TR, TR), :], xbufs[nb],
                sems[nb])
        copies[t % 2].wait()
        carry = lax.fori_loop(
            0, GPT, make_group_body(xbufs[t % 2], t * TR), carry)

    cur, k = carry[0], carry[1]
    acc = tuple(acc_ref[pl.ds(d * L, L)] for d in range(NV))

    for d in range(NV):
        stage[pl.ds(d * L, L)] = acc[d]
    pltpu.sync_copy(stage, bvals_hbm.at[wid, 1])

    @pl.when(k == 0)
    def _():
        pltpu.sync_copy(stage, bvals_hbm.at[wid, 0])

    ids = lax.iota(jnp.int32, L)
    idvec = jnp.where(ids == 0, first_id, jnp.where(ids == 1, cur, 0))
    ibuf[...] = idvec
    pltpu.sync_copy(ibuf, bids_hbm.at[wid])


_sc_pool = pl.kernel(
    _sc_body,
    out_type=(
        jax.ShapeDtypeStruct((S, D), jnp.float32),
        jax.ShapeDtypeStruct((NW, L), jnp.int32),
        jax.ShapeDtypeStruct((NW, 2, D), jnp.float32),
    ),
    mesh=plsc.VectorSubcoreMesh(core_axis_name="c", subcore_axis_name="s"),
    compiler_params=pltpu.CompilerParams(use_tc_tiling_on_sc=False),
    scratch_types=[
        pltpu.VMEM((CW + G,), jnp.int32),
        pltpu.VMEM((TR, D), jnp.float32),
        pltpu.VMEM((TR, D), jnp.float32),
        pltpu.VMEM((D,), jnp.float32),
        pltpu.VMEM((D,), jnp.float32),
        pltpu.VMEM((L,), jnp.int32),
        pltpu.VMEM((D,), jnp.float32),
        pltpu.SemaphoreType.DMA,
        pltpu.SemaphoreType.DMA,
    ],
)


def _merge_body(bids_smem, interior, bvals, out_ref):
    rows = lax.broadcasted_iota(jnp.int32, (S, D), 0)
    cov = jnp.zeros((S, D), jnp.bool_)
    for w in range(NW):
        lo = bids_smem[w, 0]
        hi = bids_smem[w, 1]
        cov = jnp.logical_or(cov, jnp.logical_and(rows > lo, rows < hi))
    out_ref[...] = jnp.where(cov, interior[...], _NEG_INF)
    for w in range(NW):
        for j in range(2):
            q = bids_smem[w, j]
            row = bvals[w, j, :].reshape(1, D)
            out_ref[pl.ds(q, 1), :] = jnp.maximum(out_ref[pl.ds(q, 1), :], row)


_merge = pl.pallas_call(
    _merge_body,
    out_shape=jax.ShapeDtypeStruct((S, D), jnp.float32),
    in_specs=[
        pl.BlockSpec(memory_space=pltpu.SMEM),
        pl.BlockSpec(memory_space=pltpu.VMEM),
        pl.BlockSpec(memory_space=pltpu.VMEM),
    ],
)


@jax.jit
def kernel(x, batch):
    interior, bids, bvals = _sc_pool(x, batch)
    return _merge(bids, interior, bvals)
